# baseline (speedup 1.0000x reference)
import functools
import jax
import jax.numpy as jnp
from jax import lax
from jax.experimental import pallas as pl
from jax.experimental.pallas import tpu as pltpu
from jax.experimental.pallas import tpu_sc as plsc

EMBED_DIM = 768
BATCH = 1024
SEQ = 77
SEQ_PAD = 80
NUM_WORKERS = 32
NB_PER_W = BATCH // NUM_WORKERS
TC_BB = 8


def _sc_gather_padded(table, idx):
  nb = idx.shape[0]
  nb_per_w = nb // NUM_WORKERS
  mesh = plsc.VectorSubcoreMesh(core_axis_name="c", subcore_axis_name="s")

  @functools.partial(
      pl.kernel,
      mesh=mesh,
      out_type=jax.ShapeDtypeStruct((nb, SEQ_PAD, EMBED_DIM), jnp.float32),
      scratch_types=[
          pltpu.VMEM((nb_per_w, SEQ_PAD), jnp.int32),
          pltpu.VMEM((SEQ_PAD, EMBED_DIM), jnp.float32),
          pltpu.VMEM((SEQ_PAD, EMBED_DIM), jnp.float32),
          pltpu.SemaphoreType.DMA,
          pltpu.SemaphoreType.DMA,
          pltpu.SemaphoreType.DMA,
          pltpu.SemaphoreType.DMA,
      ],
  )
  def k(table_hbm, idx_hbm, out_hbm, idx_v, buf0, buf1,
        gsem0, gsem1, ssem0, ssem1):
    wid = lax.axis_index("s") * 2 + lax.axis_index("c")
    base = wid * nb_per_w
    pltpu.sync_copy(idx_hbm.at[pl.ds(base, nb_per_w)], idx_v)

    bufs = (buf0, buf1)
    gsems = (gsem0, gsem1)
    ssems = (ssem0, ssem1)

    def gather(j, p):
      return pltpu.async_copy(table_hbm.at[idx_v.at[j]], bufs[p], gsems[p])

    def store(j, p):
      return pltpu.async_copy(bufs[p], out_hbm.at[base + j], ssems[p])

    gather(0, 0).wait()
    gather(1, 1)
    store(0, 0)

    def pair_body(m, carry):
      g = 1 + 2 * m
      for (j, p) in ((g, 1), (g + 1, 0)):
        q = 1 - p
        pltpu.make_async_copy(
            table_hbm.at[idx_v.at[j]], bufs[p], gsems[p]).wait()
        pltpu.make_async_copy(
            bufs[q], out_hbm.at[base + j - 1], ssems[q]).wait()
        gather(j + 1, q)
        store(j, p)
      return carry

    lax.fori_loop(0, (nb_per_w - 2) // 2, pair_body, 0, unroll=False)

    j_last = nb_per_w - 1
    pltpu.make_async_copy(
        table_hbm.at[idx_v.at[j_last]], bufs[1], gsems[1]).wait()
    pltpu.make_async_copy(
        bufs[0], out_hbm.at[base + j_last - 1], ssems[0]).wait()
    store(j_last, 1)
    pltpu.make_async_copy(
        bufs[1], out_hbm.at[base + j_last], ssems[1]).wait()

  return k(table, idx)


def _tc_unpad(padded):
  def body(in_ref, out_ref):
    out_ref[...] = in_ref[:, :SEQ, :]

  return pl.pallas_call(
      body,
      grid=(BATCH // TC_BB,),
      in_specs=[pl.BlockSpec((TC_BB, SEQ_PAD, EMBED_DIM), lambda i: (i, 0, 0))],
      out_specs=pl.BlockSpec((TC_BB, SEQ, EMBED_DIM), lambda i: (i, 0, 0)),
      out_shape=jax.ShapeDtypeStruct((BATCH, SEQ, EMBED_DIM), jnp.float32),
  )(padded)


NCHUNKS = 4
CB = BATCH // NCHUNKS


def kernel(x, token_embedding):
  idx_pad = jnp.pad(x.astype(jnp.int32), ((0, 0), (0, SEQ_PAD - SEQ)),
                    mode="edge")
  acc = jnp.zeros((BATCH, SEQ, EMBED_DIM), jnp.float32)
  for c in range(NCHUNKS):
    padded = _sc_gather_padded(
        token_embedding, lax.slice(idx_pad, (c * CB, 0), ((c + 1) * CB, SEQ_PAD)))
    acc = lax.dynamic_update_slice(acc, padded[:, :SEQ, :], (c * CB, 0, 0))
  return acc


# speedup vs baseline: 1.1504x; 1.1504x over previous
import functools
import jax
import jax.numpy as jnp
from jax import lax
from jax.experimental import pallas as pl
from jax.experimental.pallas import tpu as pltpu
from jax.experimental.pallas import tpu_sc as plsc

EMBED_DIM = 768
BATCH = 1024
SEQ = 77
SEQ_PAD = 80
NUM_WORKERS = 32
NB_PER_W = BATCH // NUM_WORKERS
TC_BB = 32


def _sc_gather_padded(table, idx):
  nb = idx.shape[0]
  nb_per_w = nb // NUM_WORKERS
  mesh = plsc.VectorSubcoreMesh(core_axis_name="c", subcore_axis_name="s")

  @functools.partial(
      pl.kernel,
      mesh=mesh,
      out_type=jax.ShapeDtypeStruct((nb, SEQ_PAD, EMBED_DIM), jnp.float32),
      scratch_types=[
          pltpu.VMEM((nb_per_w, SEQ_PAD), jnp.int32),
          pltpu.VMEM((SEQ_PAD, EMBED_DIM), jnp.float32),
          pltpu.VMEM((SEQ_PAD, EMBED_DIM), jnp.float32),
          pltpu.SemaphoreType.DMA,
          pltpu.SemaphoreType.DMA,
          pltpu.SemaphoreType.DMA,
          pltpu.SemaphoreType.DMA,
      ],
  )
  def k(table_hbm, idx_hbm, out_hbm, idx_v, buf0, buf1,
        gsem0, gsem1, ssem0, ssem1):
    wid = lax.axis_index("s") * 2 + lax.axis_index("c")
    base = wid * nb_per_w
    pltpu.sync_copy(idx_hbm.at[pl.ds(base, nb_per_w)], idx_v)

    bufs = (buf0, buf1)
    gsems = (gsem0, gsem1)
    ssems = (ssem0, ssem1)

    def gather(j, p):
      return pltpu.async_copy(table_hbm.at[idx_v.at[j]], bufs[p], gsems[p])

    def store(j, p):
      return pltpu.async_copy(bufs[p], out_hbm.at[base + j], ssems[p])

    gather(0, 0).wait()
    gather(1, 1)
    store(0, 0)

    def pair_body(m, carry):
      g = 1 + 2 * m
      for (j, p) in ((g, 1), (g + 1, 0)):
        q = 1 - p
        pltpu.make_async_copy(
            table_hbm.at[idx_v.at[j]], bufs[p], gsems[p]).wait()
        pltpu.make_async_copy(
            bufs[q], out_hbm.at[base + j - 1], ssems[q]).wait()
        gather(j + 1, q)
        store(j, p)
      return carry

    lax.fori_loop(0, (nb_per_w - 2) // 2, pair_body, 0, unroll=False)

    j_last = nb_per_w - 1
    pltpu.make_async_copy(
        table_hbm.at[idx_v.at[j_last]], bufs[1], gsems[1]).wait()
    pltpu.make_async_copy(
        bufs[0], out_hbm.at[base + j_last - 1], ssems[0]).wait()
    store(j_last, 1)
    pltpu.make_async_copy(
        bufs[1], out_hbm.at[base + j_last], ssems[1]).wait()

  return k(table, idx)


def _tc_unpad(padded):
  def body(in_ref, out_ref):
    out_ref[...] = in_ref[:, :SEQ, :]

  return pl.pallas_call(
      body,
      grid=(BATCH // TC_BB,),
      in_specs=[pl.BlockSpec((TC_BB, SEQ_PAD, EMBED_DIM), lambda i: (i, 0, 0))],
      out_specs=pl.BlockSpec((TC_BB, SEQ, EMBED_DIM), lambda i: (i, 0, 0)),
      out_shape=jax.ShapeDtypeStruct((BATCH, SEQ, EMBED_DIM), jnp.float32),
  )(padded)


NCHUNKS = 4
CB = BATCH // NCHUNKS


def kernel(x, token_embedding):
  idx_pad = jnp.pad(x.astype(jnp.int32), ((0, 0), (0, SEQ_PAD - SEQ)),
                    mode="edge")
  padded = _sc_gather_padded(token_embedding, idx_pad)
  return _tc_unpad(padded)


# speedup vs baseline: 1.5777x; 1.3714x over previous
import functools
import jax
import jax.numpy as jnp
from jax import lax
from jax.experimental import pallas as pl
from jax.experimental.pallas import tpu as pltpu
from jax.experimental.pallas import tpu_sc as plsc

EMBED_DIM = 768
BATCH = 1024
SEQ = 77
SEQ_PAD = 80
NUM_WORKERS = 32
NB_PER_W = BATCH // NUM_WORKERS


def _sc_gather_padded(table, idx):
  nb = idx.shape[0]
  nb_per_w = nb // NUM_WORKERS
  mesh = plsc.VectorSubcoreMesh(core_axis_name="c", subcore_axis_name="s")

  @functools.partial(
      pl.kernel,
      mesh=mesh,
      out_type=jax.ShapeDtypeStruct((nb, SEQ_PAD, EMBED_DIM), jnp.float32),
      scratch_types=[
          pltpu.VMEM((nb_per_w, SEQ_PAD), jnp.int32),
          pltpu.VMEM((SEQ_PAD, EMBED_DIM), jnp.float32),
          pltpu.VMEM((SEQ_PAD, EMBED_DIM), jnp.float32),
          pltpu.SemaphoreType.DMA,
          pltpu.SemaphoreType.DMA,
          pltpu.SemaphoreType.DMA,
          pltpu.SemaphoreType.DMA,
      ],
  )
  def k(table_hbm, idx_hbm, out_hbm, idx_v, buf0, buf1,
        gsem0, gsem1, ssem0, ssem1):
    wid = lax.axis_index("s") * 2 + lax.axis_index("c")
    base = wid * nb_per_w
    pltpu.sync_copy(idx_hbm.at[pl.ds(base, nb_per_w)], idx_v)

    bufs = (buf0, buf1)
    gsems = (gsem0, gsem1)
    ssems = (ssem0, ssem1)

    def gather(j, p):
      return pltpu.async_copy(table_hbm.at[idx_v.at[j]], bufs[p], gsems[p])

    def store(j, p):
      return pltpu.async_copy(bufs[p], out_hbm.at[base + j], ssems[p])

    gather(0, 0).wait()
    gather(1, 1)
    store(0, 0)

    def pair_body(m, carry):
      g = 1 + 2 * m
      for (j, p) in ((g, 1), (g + 1, 0)):
        q = 1 - p
        pltpu.make_async_copy(
            table_hbm.at[idx_v.at[j]], bufs[p], gsems[p]).wait()
        pltpu.make_async_copy(
            bufs[q], out_hbm.at[base + j - 1], ssems[q]).wait()
        gather(j + 1, q)
        store(j, p)
      return carry

    lax.fori_loop(0, (nb_per_w - 2) // 2, pair_body, 0, unroll=False)

    j_last = nb_per_w - 1
    pltpu.make_async_copy(
        table_hbm.at[idx_v.at[j_last]], bufs[1], gsems[1]).wait()
    pltpu.make_async_copy(
        bufs[0], out_hbm.at[base + j_last - 1], ssems[0]).wait()
    store(j_last, 1)
    pltpu.make_async_copy(
        bufs[1], out_hbm.at[base + j_last], ssems[1]).wait()

  return k(table, idx)


def kernel(x, token_embedding):
  idx_pad = jnp.pad(x.astype(jnp.int32), ((0, 0), (0, SEQ_PAD - SEQ)),
                    mode="edge")
  padded = _sc_gather_padded(token_embedding, idx_pad)
  return padded[:, :SEQ, :]
